# Initial kernel scaffold; baseline (speedup 1.0000x reference)
import jax
import jax.numpy as jnp
from jax.experimental import pallas as pl


def kernel(c3, c4, c5, p5_1_w, p5_1_b, p5_2_w, p5_2_b, p4_1_w, p4_1_b, p4_2_w, p4_2_b, p3_1_w, p3_1_b, p3_2_w, p3_2_b):
    raise NotImplementedError("write your pallas kernel here")


# speedup vs baseline: 1.1057x; 1.1057x over previous
import jax
import jax.numpy as jnp
from jax.experimental import pallas as pl
from jax.experimental.pallas import tpu as pltpu

_VMEM_LIMIT_BYTES = 64 * 1024 * 1024


def _fpn_level_kernel(H, W, has_skip, emit_lat, *refs):
    i = 0
    x_ref = refs[i]; i += 1
    w1_ref = refs[i]; i += 1
    b1_ref = refs[i]; i += 1
    if has_skip:
        skip_ref = refs[i]; i += 1
    w3_ref = refs[i]; i += 1
    b3_ref = refs[i]; i += 1
    if emit_lat:
        lat_ref = refs[i]; i += 1
    out_ref = refs[i]; i += 1
    xp_ref = refs[i]

    C = w1_ref.shape[1]
    HW = H * W

    lat = jnp.dot(x_ref[...], w1_ref[...], preferred_element_type=jnp.float32)
    lat = lat + b1_ref[...]

    if has_skip:
        H2, W2 = H // 2, W // 2
        prev = skip_ref[...]
        t = jnp.broadcast_to(prev.reshape(H2 * W2, 1, C), (H2 * W2, 2, C))
        t = t.reshape(H2 * W, C)
        t = jnp.broadcast_to(t.reshape(H2, 1, W, C), (H2, 2, W, C))
        lat = lat + t.reshape(HW, C)
    if emit_lat:
        lat_ref[...] = lat

    Wp = W + 8
    zc = jnp.zeros((H + 2, 8, C), jnp.bfloat16)
    xp_ref[0:1, :, :] = jnp.zeros((1, Wp, C), jnp.bfloat16)
    xp_ref[H + 1:H + 2, :, :] = jnp.zeros((1, Wp, C), jnp.bfloat16)
    xp_ref[:, 0:8, :] = zc
    xp_ref[:, W:Wp, :] = zc
    xp_ref[1:H + 1, 1:W + 1, :] = lat.astype(jnp.bfloat16).reshape(H, W, C)

    acc = None
    for k in range(9):
        dy, dx = k // 3, k % 3
        patch = xp_ref[dy:dy + H, dx:dx + W, :].reshape(HW, C)
        d = jnp.dot(patch, w3_ref[k], preferred_element_type=jnp.float32)
        acc = d if acc is None else acc + d
    out_ref[...] = acc + b3_ref[...]


def _fpn_level(x, H, W, w1, b1, skip, w3, b3, emit_lat):
    N, HW, Cin = x.shape
    C = w1.shape[1]

    in_specs = [
        pl.BlockSpec((None, HW, Cin), lambda i: (i, 0, 0)),
        pl.BlockSpec((Cin, C), lambda i: (0, 0)),
        pl.BlockSpec((1, C), lambda i: (0, 0)),
    ]
    args = [x, w1, b1]
    if skip is not None:
        in_specs.append(pl.BlockSpec((None, HW // 4, C), lambda i: (i, 0, 0)))
        args.append(skip)
    in_specs += [
        pl.BlockSpec((9, C, C), lambda i: (0, 0, 0)),
        pl.BlockSpec((1, C), lambda i: (0, 0)),
    ]
    args += [w3, b3]

    out_shape = [jax.ShapeDtypeStruct((N, HW, C), jnp.float32)]
    out_specs = [pl.BlockSpec((None, HW, C), lambda i: (i, 0, 0))]
    if emit_lat:
        out_shape = out_shape + out_shape
        out_specs = out_specs + out_specs

    import functools
    body = functools.partial(_fpn_level_kernel, H, W, skip is not None, emit_lat)
    res = pl.pallas_call(
        body,
        grid=(N,),
        in_specs=in_specs,
        out_specs=tuple(out_specs),
        out_shape=tuple(out_shape),
        scratch_shapes=[pltpu.VMEM((H + 2, W + 8, C), jnp.bfloat16)],
        compiler_params=pltpu.CompilerParams(
            dimension_semantics=("parallel",),
            vmem_limit_bytes=_VMEM_LIMIT_BYTES,
        ),
    )(*args)
    if emit_lat:
        return res[0], res[1]
    return None, res[0]


def kernel(c3, c4, c5,
           p5_1_w, p5_1_b, p5_2_w, p5_2_b,
           p4_1_w, p4_1_b, p4_2_w, p4_2_b,
           p3_1_w, p3_1_b, p3_2_w, p3_2_b):
    N = c3.shape[0]
    bf = jnp.bfloat16

    def to_rows(x):
        n, c, h, w = x.shape
        return jnp.transpose(x.astype(bf), (0, 2, 3, 1)).reshape(n, h * w, c)

    x5, x4, x3 = to_rows(c5), to_rows(c4), to_rows(c3)
    h5, w5 = c5.shape[2], c5.shape[3]
    h4, w4 = c4.shape[2], c4.shape[3]
    h3, w3_ = c3.shape[2], c3.shape[3]

    p5_lat, p5_out = _fpn_level(x5, h5, w5, p5_1_w.astype(bf), p5_1_b,
                                None, p5_2_w.astype(bf), p5_2_b, emit_lat=True)
    p4_lat, p4_out = _fpn_level(x4, h4, w4, p4_1_w.astype(bf), p4_1_b,
                                p5_lat, p4_2_w.astype(bf), p4_2_b, emit_lat=True)
    _, p3_out = _fpn_level(x3, h3, w3_, p3_1_w.astype(bf), p3_1_b,
                           p4_lat, p3_2_w.astype(bf), p3_2_b, emit_lat=False)

    def to_nchw(o, h, w):
        return jnp.transpose(o.reshape(N, h, w, o.shape[-1]), (0, 3, 1, 2))

    return [to_nchw(p3_out, h3, w3_), to_nchw(p4_out, h4, w4),
            to_nchw(p5_out, h5, w5)]
